# baseline (speedup 1.0000x reference)
import functools

import jax
import jax.numpy as jnp
from jax import lax
from jax.experimental import pallas as pl
from jax.experimental.pallas import tpu as pltpu
from jax.experimental.pallas import tpu_sc as plsc

T = 2048
D = 1024
DFF = 768
E = 8
K = 2
NPAIR = T * K
BT = 256
NBUF = NPAIR + E * BT
NT = NBUF // BT - 1

NC = 2
NS = 16
LANES = 16
PAIRS_PER_SUB = NPAIR // NS
ROWS_PER_TILE = NBUF // (NC * NS)
GCHUNK = 64
TOK_PER_TILE = T // (NC * NS)
CCHUNK = 32
RB = 128


def _router_body(x_ref, gw_ref, pos_ref, w_ref, texp_ref):
    x = x_ref[...]
    gw = gw_ref[...]
    logits = lax.dot_general(
        x, gw, (((1,), (1,)), ((), ())), preferred_element_type=jnp.float32
    )
    iota = lax.broadcasted_iota(jnp.int32, (T, E), 1)
    m1 = jnp.max(logits, axis=1, keepdims=True)
    i1 = jnp.min(jnp.where(logits == m1, iota, E), axis=1, keepdims=True)
    oh1 = (iota == i1).astype(jnp.float32)
    lm = jnp.where(iota == i1, -jnp.inf, logits)
    m2 = jnp.max(lm, axis=1, keepdims=True)
    i2 = jnp.min(jnp.where(lm == m2, iota, E), axis=1, keepdims=True)
    oh2 = (iota == i2).astype(jnp.float32)
    e2 = jnp.exp(m2 - m1)
    sden = 1.0 + e2
    w_ref[...] = jnp.concatenate([1.0 / sden, e2 / sden], axis=1)

    ohsum = oh1 + oh2

    r_i = lax.broadcasted_iota(jnp.int32, (RB, RB), 0)
    c_i = lax.broadcasted_iota(jnp.int32, (RB, RB), 1)
    ltri = (c_i < r_i).astype(jnp.float32)
    nblk = T // RB
    s_blocks = []
    bsums = []
    for b in range(nblk):
        blk = ohsum[b * RB:(b + 1) * RB, :]
        s_blocks.append(
            lax.dot_general(ltri, blk, (((1,), (0,)), ((), ())),
                            preferred_element_type=jnp.float32))
        bsums.append(jnp.sum(blk, axis=0, keepdims=True))
    bs = jnp.concatenate(bsums, axis=0)
    rb_i = lax.broadcasted_iota(jnp.int32, (nblk, nblk), 0)
    cb_i = lax.broadcasted_iota(jnp.int32, (nblk, nblk), 1)
    ltri_b = (cb_i < rb_i).astype(jnp.float32)
    bpre = lax.dot_general(ltri_b, bs, (((1,), (0,)), ((), ())),
                           preferred_element_type=jnp.float32)
    s_excl = jnp.concatenate(
        [s_blocks[b] + bpre[b:b + 1, :] for b in range(nblk)], axis=0)

    ones_t = jnp.ones((T, 1), jnp.float32)
    counts_col = lax.dot_general(ohsum, ones_t, (((0,), (0,)), ((), ())),
                                 preferred_element_type=jnp.float32)
    tiles_col = jnp.floor((counts_col + (BT - 1)) * (1.0 / BT))
    e_r = lax.broadcasted_iota(jnp.int32, (E, E), 0)
    e_c = lax.broadcasted_iota(jnp.int32, (E, E), 1)
    ltri_e = (e_c < e_r).astype(jnp.float32)
    base_col = lax.dot_general(ltri_e, tiles_col, (((1,), (0,)), ((), ())),
                               preferred_element_type=jnp.float32)
    base_row = lax.dot_general(tiles_col, ltri_e, (((0,), (1,)), ((), ())),
                               preferred_element_type=jnp.float32)
    base_slot_row = base_row * float(BT)

    pos1 = jnp.sum(oh1 * (base_slot_row + s_excl), axis=1, keepdims=True)
    pos2 = jnp.sum(oh2 * (base_slot_row + s_excl + oh1), axis=1, keepdims=True)
    pos_ref[...] = jnp.concatenate([pos1, pos2], axis=1).astype(jnp.int32)

    end_col = base_col + tiles_col
    t_iota = lax.broadcasted_iota(jnp.int32, (E, 2 * LANES), 1).astype(
        jnp.float32)
    ge = (t_iota >= end_col).astype(jnp.float32)
    texp = jnp.sum(ge, axis=0, keepdims=True)
    texp_ref[...] = jnp.minimum(texp, float(E - 1)).astype(jnp.int32)


def _scatter_body(pos_hbm, w_hbm, gidx_hbm, wslot_hbm, pos_v, w_v, tok_v):
    c = lax.axis_index("c")
    s = lax.axis_index("s")
    wid = s * NC + c
    iota16 = lax.broadcasted_iota(jnp.int32, (LANES,), 0)

    def splat(v):
        return jnp.full((LANES,), v, jnp.int32)

    one_vec = splat(1)

    pltpu.sync_copy(pos_hbm.at[wid], pos_v)
    pltpu.sync_copy(w_hbm.at[wid], w_v)
    for v in range(8):
        base_pair = wid * 128 + v * LANES
        tok_v[0, pl.ds(v * LANES, LANES)] = (iota16 + splat(base_pair)) >> one_vec

    pltpu.sync_copy(tok_v.at[0], gidx_hbm.at[pos_v.at[0]])
    pltpu.sync_copy(w_v.at[0], wslot_hbm.at[pos_v.at[0]])


def _scatter(pos_r, w_r):
    mesh = plsc.VectorSubcoreMesh(core_axis_name="c", subcore_axis_name="s")
    return pl.kernel(
        _scatter_body,
        out_type=(
            jax.ShapeDtypeStruct((NBUF,), jnp.int32),
            jax.ShapeDtypeStruct((NBUF,), jnp.float32),
        ),
        mesh=mesh,
        scratch_types=[
            pltpu.VMEM((1, 128), jnp.int32),
            pltpu.VMEM((1, 128), jnp.float32),
            pltpu.VMEM((1, 128), jnp.int32),
        ],
    )(pos_r, w_r)


def _gather_body(gidx_hbm, x_hbm, xg_hbm, gidx_v, rows_v, sem):
    c = lax.axis_index("c")
    s = lax.axis_index("s")
    wid = s * NC + c

    def splat(v):
        return jnp.full((LANES,), v, jnp.int32)

    base = wid * ROWS_PER_TILE
    nch = ROWS_PER_TILE // GCHUNK
    for g in range(nch):
        pltpu.sync_copy(
            gidx_hbm.at[pl.ds(base + g * GCHUNK, GCHUNK)], gidx_v.at[g])
    for g in range(nch):
        for i in range(GCHUNK // LANES):
            v = gidx_v[g, pl.ds(i * LANES, LANES)]
            gidx_v[g, pl.ds(i * LANES, LANES)] = jnp.minimum(
                jnp.maximum(v, splat(0)), splat(T - 1)
            )
    for g in range(nch):
        pltpu.async_copy(x_hbm.at[gidx_v.at[g]], rows_v, sem).wait()
        pltpu.sync_copy(rows_v, xg_hbm.at[pl.ds(base + g * GCHUNK, GCHUNK)])


def _gather(gidx, x):
    mesh = plsc.VectorSubcoreMesh(core_axis_name="c", subcore_axis_name="s")
    return pl.kernel(
        _gather_body,
        out_type=jax.ShapeDtypeStruct((NBUF, D), jnp.float32),
        mesh=mesh,
        scratch_types=[
            pltpu.VMEM((ROWS_PER_TILE // GCHUNK, GCHUNK), jnp.int32),
            pltpu.VMEM((GCHUNK, D), jnp.float32),
            pltpu.SemaphoreType.DMA,
        ],
    )(gidx, x)


def _ffn_body(texp_ref, xg_ref, ws_ref, wg_ref, wu_ref, wd_ref, out_ref):
    x = xg_ref[...].astype(jnp.bfloat16)
    wg = wg_ref[0].astype(jnp.bfloat16)
    wu = wu_ref[0].astype(jnp.bfloat16)
    wd = wd_ref[0].astype(jnp.bfloat16)
    g = lax.dot_general(
        x, wg, (((1,), (1,)), ((), ())), preferred_element_type=jnp.float32
    )
    u = lax.dot_general(
        x, wu, (((1,), (1,)), ((), ())), preferred_element_type=jnp.float32
    )
    act = (g * (1.0 / (1.0 + jnp.exp(-g)))) * u * ws_ref[...]
    out_ref[...] = jnp.dot(
        act.astype(jnp.bfloat16), wd, preferred_element_type=jnp.float32
    )


def _ffn(texp, xg, wslot2d, w_gate, w_up, w_down):
    grid_spec = pltpu.PrefetchScalarGridSpec(
        num_scalar_prefetch=1,
        grid=(NT,),
        in_specs=[
            pl.BlockSpec((BT, D), lambda i, texp: (i, 0)),
            pl.BlockSpec((BT, 1), lambda i, texp: (i, 0)),
            pl.BlockSpec((1, DFF, D), lambda i, texp: (texp[i], 0, 0)),
            pl.BlockSpec((1, DFF, D), lambda i, texp: (texp[i], 0, 0)),
            pl.BlockSpec((1, DFF, D), lambda i, texp: (texp[i], 0, 0)),
        ],
        out_specs=pl.BlockSpec((BT, D), lambda i, texp: (i, 0)),
    )
    return pl.pallas_call(
        _ffn_body,
        grid_spec=grid_spec,
        out_shape=jax.ShapeDtypeStruct((NBUF, D), jnp.float32),
        compiler_params=pltpu.CompilerParams(
            dimension_semantics=("arbitrary",),
        ),
    )(texp, xg, wslot2d, w_gate, w_up, w_down)


def _combine_body(pos0_hbm, pos1_hbm, ob_hbm, out_hbm, idx0_v, idx1_v,
                  buf1, buf2, out_v, sem0, sem1):
    c = lax.axis_index("c")
    s = lax.axis_index("s")
    wid = s * NC + c
    for ch in range(TOK_PER_TILE // CCHUNK):
        tb = wid * TOK_PER_TILE + ch * CCHUNK
        pltpu.sync_copy(pos0_hbm.at[pl.ds(tb, CCHUNK)], idx0_v)
        pltpu.sync_copy(pos1_hbm.at[pl.ds(tb, CCHUNK)], idx1_v)
        cp0 = pltpu.async_copy(ob_hbm.at[idx0_v], buf1, sem0)
        cp1 = pltpu.async_copy(ob_hbm.at[idx1_v], buf2, sem1)
        cp0.wait()
        cp1.wait()

        def body(i, _):
            for j in range(D // LANES):
                sl = pl.ds(j * LANES, LANES)
                out_v[i, sl] = buf1[i, sl] + buf2[i, sl]
            return 0

        lax.fori_loop(0, CCHUNK, body, 0)
        pltpu.sync_copy(out_v, out_hbm.at[pl.ds(tb, CCHUNK)])


def _combine(pos0, pos1, ob):
    mesh = plsc.VectorSubcoreMesh(core_axis_name="c", subcore_axis_name="s")
    return pl.kernel(
        _combine_body,
        out_type=jax.ShapeDtypeStruct((T, D), jnp.float32),
        mesh=mesh,
        scratch_types=[
            pltpu.VMEM((CCHUNK,), jnp.int32),
            pltpu.VMEM((CCHUNK,), jnp.int32),
            pltpu.VMEM((CCHUNK, D), jnp.float32),
            pltpu.VMEM((CCHUNK, D), jnp.float32),
            pltpu.VMEM((CCHUNK, D), jnp.float32),
            pltpu.SemaphoreType.DMA,
            pltpu.SemaphoreType.DMA,
        ],
    )(pos0, pos1, ob)


@jax.jit
def kernel(hidden_states, gathered_experts_out_buf, gate_w, w_gate, w_up, w_down):
    x = hidden_states.reshape(T, D)
    pos2d, w2, texp2d = pl.pallas_call(
        _router_body,
        out_shape=(
            jax.ShapeDtypeStruct((T, K), jnp.int32),
            jax.ShapeDtypeStruct((T, K), jnp.float32),
            jax.ShapeDtypeStruct((1, 2 * LANES), jnp.int32),
        ),
    )(x, gate_w)

    pos_r = pos2d.reshape(NC * NS, 1, 128)
    w_r = w2.reshape(NC * NS, 1, 128)
    gidx, wslot = _scatter(pos_r, w_r)
    xg = _gather(gidx, x)

    texp = texp2d.reshape(2 * LANES)
    ob = _ffn(texp, xg, wslot.reshape(NBUF, 1), w_gate, w_up, w_down)

    final = _combine(pos2d[:, 0], pos2d[:, 1], ob)
    return final.reshape(hidden_states.shape)
